# baseline (speedup 1.0000x reference)
import jax
import jax.numpy as jnp
from jax import lax
from jax.experimental import pallas as pl
from jax.experimental.pallas import tpu as pltpu
from jax.experimental.pallas import tpu_sc as plsc

NUM_DOCS = 100000
VOCAB = 100000
D_DOC = 128
D_WORD = 64
BATCH = 1024
N_GRAM = 5
Z = 10.0
LAMB = 0.01

ROW = VOCAB
NC, NS, L = 2, 16, 16
NW = NC * NS
DOCS_PER_W = BATCH // NW
WORD_ROWS_PER_W = D_WORD // NW
SUM_UNROLL = 8


CHUNKS = ((0, 24960), (24960, 24960), (49920, 24960), (74880, 25088))
N_CHUNK = len(CHUNKS)
TAIL_OFF = 99968
TAIL = ROW - TAIL_OFF


def _sc_body(rdT_h, rv_h, rvt_h, didx_h, wids_h, doc_h, wpT_h, part_h,
             row_v, didx_v, wids_v, out_v, rows_v, acc_v, sem, *csems):
    wid = lax.axis_index("s") * NC + lax.axis_index("c")

    def _fire_row(r):
        descs = [
            pltpu.async_copy(rv_h.at[r].at[pl.ds(off, sz)],
                             row_v.at[pl.ds(off, sz)], csems[c])
            for c, (off, sz) in enumerate(CHUNKS)
        ]
        descs.append(pltpu.async_copy(rvt_h.at[pl.ds(r * TAIL, TAIL)],
                                      row_v.at[pl.ds(TAIL_OFF, TAIL)],
                                      csems[N_CHUNK]))
        return descs

    r0 = wid * WORD_ROWS_PER_W
    descs = _fire_row(r0)

    base = wid * DOCS_PER_W
    pltpu.sync_copy(didx_h.at[pl.ds(base, DOCS_PER_W)], didx_v)
    pltpu.async_copy(rdT_h.at[didx_v], rows_v, sem).wait()
    pltpu.sync_copy(rows_v, doc_h.at[pl.ds(base, DOCS_PER_W), :])
    pltpu.sync_copy(wids_h, wids_v)

    accs = tuple(jnp.zeros((L,), jnp.float32) for _ in range(SUM_UNROLL))

    def _sum_chunk(accs, chunk_base, chunk_sz):
        def sbody(i, a):
            off = chunk_base + i * (SUM_UNROLL * L)
            return tuple(
                a[u] + row_v[pl.ds(off + u * L, L)] for u in range(SUM_UNROLL)
            )
        return lax.fori_loop(0, chunk_sz // (SUM_UNROLL * L), sbody, accs)

    for k in range(WORD_ROWS_PER_W):
        r = wid * WORD_ROWS_PER_W + k
        for c, (off, sz) in enumerate(CHUNKS):
            descs[c].wait()
            accs = _sum_chunk(accs, off, sz)
        descs[N_CHUNK].wait()
        accs = (accs[0] + row_v[pl.ds(TAIL_OFF, L)],
                accs[1] + row_v[pl.ds(TAIL_OFF + L, L)]) + accs[2:]

        def wbody(c, _):
            s = jnp.zeros((L,), jnp.float32)
            for g in range(N_GRAM):
                idx = wids_v[g, pl.ds(c * L, L)]
                s = s + plsc.load_gather(row_v, [idx])
            out_v[pl.ds(c * L, L)] = s * (1.0 / N_GRAM)
            return 0
        lax.fori_loop(0, BATCH // L, wbody, 0)
        if k + 1 < WORD_ROWS_PER_W:
            descs = _fire_row(r + 1)
        pltpu.sync_copy(out_v, wpT_h.at[r])

    acc = accs[0]
    for u in range(1, SUM_UNROLL):
        acc = acc + accs[u]
    acc_v[...] = acc
    pltpu.sync_copy(acc_v, part_h.at[wid])


def _make_sc_kernel():
    return pl.kernel(
        _sc_body,
        out_type=(
            jax.ShapeDtypeStruct((BATCH, D_DOC), jnp.float32),
            jax.ShapeDtypeStruct((D_WORD, BATCH), jnp.float32),
            jax.ShapeDtypeStruct((NW, L), jnp.float32),
        ),
        mesh=plsc.VectorSubcoreMesh(core_axis_name="c", subcore_axis_name="s",
                                    num_cores=NC, num_subcores=NS),
        compiler_params=pltpu.CompilerParams(needs_layout_passes=False),
        scratch_types=[
            pltpu.VMEM((ROW,), jnp.float32),
            pltpu.VMEM((DOCS_PER_W,), jnp.int32),
            pltpu.VMEM((N_GRAM, BATCH), jnp.int32),
            pltpu.VMEM((BATCH,), jnp.float32),
            pltpu.VMEM((DOCS_PER_W, D_DOC), jnp.float32),
            pltpu.VMEM((L,), jnp.float32),
            pltpu.SemaphoreType.DMA,
            *[pltpu.SemaphoreType.DMA for _ in range(N_CHUNK + 1)],
        ],
    )


RD_CHUNK = 10000


def _rdsum_body(x_ref, o_ref):
    @pl.when(pl.program_id(0) == 0)
    def _():
        o_ref[...] = jnp.zeros_like(o_ref)
    o_ref[...] += jnp.sum(x_ref[...], axis=0, keepdims=True)


def _tc_rdsum(rdT):
    return pl.pallas_call(
        _rdsum_body,
        grid=(NUM_DOCS // RD_CHUNK,),
        in_specs=[pl.BlockSpec((RD_CHUNK, D_DOC), lambda i: (i, 0))],
        out_specs=pl.BlockSpec((1, D_DOC), lambda i: (0, 0)),
        out_shape=jax.ShapeDtypeStruct((1, D_DOC), jnp.float32),
    )(rdT)


B_BLK = 128


def _tc_body(wpT_ref, doc_ref, projT_ref, beta_ref, part_ref, rdp_ref,
             out_ref, loss_ref):
    wp = wpT_ref[...].T
    doc = doc_ref[...]
    projT = projT_ref[...]
    beta3 = beta_ref[...].T.reshape(1, D_WORD, 1)

    inv_n = lax.rsqrt(jnp.sum(wp * wp, axis=1, keepdims=True))
    xn = wp * inv_n

    mu = jnp.dot(xn, projT, preferred_element_type=jnp.float32,
                 precision=lax.Precision.HIGHEST) * (1.0 / D_WORD)
    sq = jnp.dot(xn * xn, projT * projT, preferred_element_type=jnp.float32,
                 precision=lax.Precision.HIGHEST)
    var = (sq - D_WORD * mu * mu) * (1.0 / (D_WORD - 1))
    inv = lax.rsqrt(jnp.sqrt(var))

    docn = doc * (-1.4426950408889634)
    fx = projT[None, :, :] * xn[:, :, None]
    t = jnp.clip((fx - mu[:, None, :]) * inv[:, None, :] + beta3, -1.0, 1.0)
    e = lax.exp2(t * docn[:, None, :])
    out_ref[...] = lax.log(1.0 + e) * (-Z)

    total = jnp.sum(part_ref[...]) + jnp.sum(rdp_ref[...]) + jnp.sum(projT)
    loss_ref[...] = jnp.broadcast_to((LAMB / (2.0 * BATCH)) * total, (1, 1))


def _tc_dense(wpT, doc, projT, betaT, parts, rd_part):
    return pl.pallas_call(
        _tc_body,
        grid=(BATCH // B_BLK,),
        in_specs=[
            pl.BlockSpec((D_WORD, B_BLK), lambda i: (0, i)),
            pl.BlockSpec((B_BLK, D_DOC), lambda i: (i, 0)),
            pl.BlockSpec((D_WORD, D_DOC), lambda i: (0, 0)),
            pl.BlockSpec((1, D_WORD), lambda i: (0, 0)),
            pl.BlockSpec((NW, L), lambda i: (0, 0)),
            pl.BlockSpec((1, D_DOC), lambda i: (0, 0)),
        ],
        out_specs=[
            pl.BlockSpec((B_BLK, D_WORD, D_DOC), lambda i: (i, 0, 0)),
            pl.BlockSpec((1, 1), lambda i: (0, 0)),
        ],
        out_shape=[
            jax.ShapeDtypeStruct((BATCH, D_WORD, D_DOC), jnp.float32),
            jax.ShapeDtypeStruct((1, 1), jnp.float32),
        ],
    )(wpT, doc, projT, betaT, parts, rd_part)


def kernel(rd, rv, proj, beta, doc_idx, word_ids):
    rdT = jnp.swapaxes(rd, 0, 1)
    projT = jnp.swapaxes(proj, 0, 1)
    didx = doc_idx.astype(jnp.int32)
    widsT = jnp.swapaxes(word_ids.astype(jnp.int32), 0, 1)
    rv_tail = lax.slice(rv, (0, TAIL_OFF), (D_WORD, ROW)).reshape(
        D_WORD * TAIL)
    doc, wpT, parts = _make_sc_kernel()(rdT, rv, rv_tail, didx, widsT)
    rd_part = _tc_rdsum(rdT)
    out3, loss = _tc_dense(wpT, doc, projT, jnp.swapaxes(beta, 0, 1),
                           parts, rd_part)
    return jnp.swapaxes(out3, 1, 2), loss[0, 0]


# speedup vs baseline: 1.0116x; 1.0116x over previous
import jax
import jax.numpy as jnp
from jax import lax
from jax.experimental import pallas as pl
from jax.experimental.pallas import tpu as pltpu
from jax.experimental.pallas import tpu_sc as plsc

NUM_DOCS = 100000
VOCAB = 100000
D_DOC = 128
D_WORD = 64
BATCH = 1024
N_GRAM = 5
Z = 10.0
LAMB = 0.01

ROW = VOCAB
NC, NS, L = 2, 16, 16
NW = NC * NS
DOCS_PER_W = BATCH // NW
WORD_ROWS_PER_W = D_WORD // NW
SUM_UNROLL = 8


CHUNKS = ((0, 33280), (33280, 33280), (66560, 33408))
N_CHUNK = len(CHUNKS)
TAIL_OFF = 99968
TAIL = ROW - TAIL_OFF


def _sc_body(rdT_h, rv_h, rvt_h, didx_h, wids_h, doc_h, wpT_h, part_h,
             row_v, didx_v, wids_v, out_v, rows_v, acc_v, sem, *csems):
    wid = lax.axis_index("s") * NC + lax.axis_index("c")

    def _fire_row(r):
        descs = [
            pltpu.async_copy(rv_h.at[r].at[pl.ds(off, sz)],
                             row_v.at[pl.ds(off, sz)], csems[c])
            for c, (off, sz) in enumerate(CHUNKS)
        ]
        descs.append(pltpu.async_copy(rvt_h.at[pl.ds(r * TAIL, TAIL)],
                                      row_v.at[pl.ds(TAIL_OFF, TAIL)],
                                      csems[N_CHUNK]))
        return descs

    r0 = wid * WORD_ROWS_PER_W
    descs = _fire_row(r0)

    base = wid * DOCS_PER_W
    pltpu.sync_copy(didx_h.at[pl.ds(base, DOCS_PER_W)], didx_v)
    pltpu.async_copy(rdT_h.at[didx_v], rows_v, sem).wait()
    pltpu.sync_copy(rows_v, doc_h.at[pl.ds(base, DOCS_PER_W), :])
    pltpu.sync_copy(wids_h, wids_v)

    accs = tuple(jnp.zeros((L,), jnp.float32) for _ in range(SUM_UNROLL))

    def _sum_chunk(accs, chunk_base, chunk_sz):
        def sbody(i, a):
            off = chunk_base + i * (SUM_UNROLL * L)
            return tuple(
                a[u] + row_v[pl.ds(off + u * L, L)] for u in range(SUM_UNROLL)
            )
        return lax.fori_loop(0, chunk_sz // (SUM_UNROLL * L), sbody, accs)

    for k in range(WORD_ROWS_PER_W):
        r = wid * WORD_ROWS_PER_W + k
        for c, (off, sz) in enumerate(CHUNKS):
            descs[c].wait()
            accs = _sum_chunk(accs, off, sz)
        descs[N_CHUNK].wait()
        accs = (accs[0] + row_v[pl.ds(TAIL_OFF, L)],
                accs[1] + row_v[pl.ds(TAIL_OFF + L, L)]) + accs[2:]

        def wbody(c, _):
            s = jnp.zeros((L,), jnp.float32)
            for g in range(N_GRAM):
                idx = wids_v[pl.ds(g * BATCH + c * L, L)]
                s = s + plsc.load_gather(row_v, [idx])
            out_v[pl.ds(c * L, L)] = s * (1.0 / N_GRAM)
            return 0
        lax.fori_loop(0, BATCH // L, wbody, 0)
        if k + 1 < WORD_ROWS_PER_W:
            descs = _fire_row(r + 1)
        pltpu.sync_copy(out_v, wpT_h.at[r])

    acc = accs[0]
    for u in range(1, SUM_UNROLL):
        acc = acc + accs[u]
    acc_v[...] = acc
    pltpu.sync_copy(acc_v, part_h.at[wid])


def _make_sc_kernel():
    return pl.kernel(
        _sc_body,
        out_type=(
            jax.ShapeDtypeStruct((BATCH, D_DOC), jnp.float32),
            jax.ShapeDtypeStruct((D_WORD, BATCH), jnp.float32),
            jax.ShapeDtypeStruct((NW, L), jnp.float32),
        ),
        mesh=plsc.VectorSubcoreMesh(core_axis_name="c", subcore_axis_name="s",
                                    num_cores=NC, num_subcores=NS),
        compiler_params=pltpu.CompilerParams(needs_layout_passes=False),
        scratch_types=[
            pltpu.VMEM((ROW,), jnp.float32),
            pltpu.VMEM((DOCS_PER_W,), jnp.int32),
            pltpu.VMEM((N_GRAM * BATCH,), jnp.int32),
            pltpu.VMEM((BATCH,), jnp.float32),
            pltpu.VMEM((DOCS_PER_W, D_DOC), jnp.float32),
            pltpu.VMEM((L,), jnp.float32),
            pltpu.SemaphoreType.DMA,
            *[pltpu.SemaphoreType.DMA for _ in range(N_CHUNK + 1)],
        ],
    )


RD_CHUNK = 10000


def _rdsum_body(x_ref, o_ref):
    @pl.when(pl.program_id(0) == 0)
    def _():
        o_ref[...] = jnp.zeros_like(o_ref)
    o_ref[...] += jnp.sum(x_ref[...], axis=0, keepdims=True)


def _tc_rdsum(rdT):
    return pl.pallas_call(
        _rdsum_body,
        grid=(NUM_DOCS // RD_CHUNK,),
        in_specs=[pl.BlockSpec((RD_CHUNK, D_DOC), lambda i: (i, 0))],
        out_specs=pl.BlockSpec((1, D_DOC), lambda i: (0, 0)),
        out_shape=jax.ShapeDtypeStruct((1, D_DOC), jnp.float32),
    )(rdT)


B_BLK = 128


def _tc_body(wpT_ref, doc_ref, projT_ref, beta_ref, part_ref, rdp_ref,
             out_ref, loss_ref):
    wp = wpT_ref[...].T
    doc = doc_ref[...]
    projT = projT_ref[...]
    beta3 = beta_ref[...].T.reshape(1, D_WORD, 1)

    inv_n = lax.rsqrt(jnp.sum(wp * wp, axis=1, keepdims=True))
    xn = wp * inv_n

    mu = jnp.dot(xn, projT, preferred_element_type=jnp.float32,
                 precision=lax.Precision.HIGHEST) * (1.0 / D_WORD)
    sq = jnp.dot(xn * xn, projT * projT, preferred_element_type=jnp.float32,
                 precision=lax.Precision.HIGHEST)
    var = (sq - D_WORD * mu * mu) * (1.0 / (D_WORD - 1))
    inv = lax.rsqrt(jnp.sqrt(var))

    docn = doc * (-1.4426950408889634)
    fx = projT[None, :, :] * xn[:, :, None]
    t = jnp.clip((fx - mu[:, None, :]) * inv[:, None, :] + beta3, -1.0, 1.0)
    e = lax.exp2(t * docn[:, None, :])
    out_ref[...] = lax.log(1.0 + e) * (-Z)

    total = jnp.sum(part_ref[...]) + jnp.sum(rdp_ref[...]) + jnp.sum(projT)
    loss_ref[...] = jnp.broadcast_to((LAMB / (2.0 * BATCH)) * total, (1, 1))


def _tc_dense(wpT, doc, projT, betaT, parts, rd_part):
    return pl.pallas_call(
        _tc_body,
        grid=(BATCH // B_BLK,),
        in_specs=[
            pl.BlockSpec((D_WORD, B_BLK), lambda i: (0, i)),
            pl.BlockSpec((B_BLK, D_DOC), lambda i: (i, 0)),
            pl.BlockSpec((D_WORD, D_DOC), lambda i: (0, 0)),
            pl.BlockSpec((1, D_WORD), lambda i: (0, 0)),
            pl.BlockSpec((NW, L), lambda i: (0, 0)),
            pl.BlockSpec((1, D_DOC), lambda i: (0, 0)),
        ],
        out_specs=[
            pl.BlockSpec((B_BLK, D_WORD, D_DOC), lambda i: (i, 0, 0)),
            pl.BlockSpec((1, 1), lambda i: (0, 0)),
        ],
        out_shape=[
            jax.ShapeDtypeStruct((BATCH, D_WORD, D_DOC), jnp.float32),
            jax.ShapeDtypeStruct((1, 1), jnp.float32),
        ],
    )(wpT, doc, projT, betaT, parts, rd_part)


def kernel(rd, rv, proj, beta, doc_idx, word_ids):
    rdT = jnp.swapaxes(rd, 0, 1)
    projT = jnp.swapaxes(proj, 0, 1)
    didx = doc_idx.astype(jnp.int32)
    wids = word_ids.astype(jnp.int32).T.reshape(N_GRAM * BATCH)
    rv_tail = lax.slice(rv, (0, TAIL_OFF), (D_WORD, ROW)).reshape(
        D_WORD * TAIL)
    doc, wpT, parts = _make_sc_kernel()(rdT, rv, rv_tail, didx, wids)
    rd_part = _tc_rdsum(rdT)
    out3, loss = _tc_dense(wpT, doc, projT, jnp.swapaxes(beta, 0, 1),
                           parts, rd_part)
    return jnp.swapaxes(out3, 1, 2), loss[0, 0]
